# Initial kernel scaffold; baseline (speedup 1.0000x reference)
import functools

import jax
import jax.numpy as jnp
from jax import lax
from jax.experimental import pallas as pl
from jax.experimental.pallas import tpu as pltpu
from jax.experimental.pallas import tpu_sc as plsc

_NC = 2
_NS = 16
_NW = _NC * _NS
_C = 80


def _sc_mesh():
    return plsc.VectorSubcoreMesh(core_axis_name="c", subcore_axis_name="s")


def _worker_id():
    c = lax.axis_index("c")
    s = lax.axis_index("s")
    return c * _NS + s, c, s


def _degrees(src, dst, n_nodes):
    e = src.shape[0]
    assert e % _NW == 0
    epw = e // _NW
    n_chunks = epw // _C
    assert n_chunks * _C == epw
    rows_per_sub = n_nodes // _NS
    assert rows_per_sub * _NS == n_nodes

    ones = jnp.ones((_C, 16), dtype=jnp.float32)
    zeros = jnp.zeros((rows_per_sub, 16), dtype=jnp.float32)

    @functools.partial(
        pl.kernel,
        out_type=[
            jax.ShapeDtypeStruct((_NC, n_nodes, 16), jnp.float32),
            jax.ShapeDtypeStruct((_NC, n_nodes, 16), jnp.float32),
        ],
        mesh=_sc_mesh(),
        scratch_types=[
            pltpu.VMEM((1, _C), jnp.int32),
            pltpu.VMEM((1, _C), jnp.int32),
            pltpu.VMEM((_C, 16), jnp.float32),
            pltpu.VMEM_SHARED((n_nodes, 16), jnp.float32),
            pltpu.VMEM_SHARED((n_nodes, 16), jnp.float32),
        ],
    )
    def k(src_h, dst_h, ones_h, zeros_h, dego_h, degi_h,
          sidx, didx, ones_v, dego_sh, degi_sh):
        w, c, s = _worker_id()
        r0 = s * rows_per_sub
        pltpu.sync_copy(zeros_h, dego_sh.at[pl.ds(r0, rows_per_sub)])
        pltpu.sync_copy(zeros_h, degi_sh.at[pl.ds(r0, rows_per_sub)])
        pltpu.sync_copy(ones_h, ones_v)
        plsc.subcore_barrier()

        base = w * epw

        def body(j, carry):
            off = base + j * _C
            pltpu.sync_copy(src_h.at[pl.ds(off, _C)], sidx.at[0])
            pltpu.sync_copy(dst_h.at[pl.ds(off, _C)], didx.at[0])
            pltpu.sync_copy(ones_v, dego_sh.at[sidx.at[0]], add=True)
            pltpu.sync_copy(ones_v, degi_sh.at[didx.at[0]], add=True)
            return carry

        lax.fori_loop(0, n_chunks, body, 0)
        plsc.subcore_barrier()
        pltpu.sync_copy(dego_sh.at[pl.ds(r0, rows_per_sub)],
                        dego_h.at[c, pl.ds(r0, rows_per_sub)])
        pltpu.sync_copy(degi_sh.at[pl.ds(r0, rows_per_sub)],
                        degi_h.at[c, pl.ds(r0, rows_per_sub)])

    return k(src, dst, ones, zeros)


def _spmm(x, src, dst):
    n_nodes, d = x.shape
    e = src.shape[0]
    epw = e // _NW
    n_chunks = epw // _C
    rows_per_sub = n_nodes // _NS
    zeros = jnp.zeros((rows_per_sub, d), dtype=jnp.float32)

    @functools.partial(
        pl.kernel,
        out_type=jax.ShapeDtypeStruct((_NC, n_nodes, d), jnp.float32),
        mesh=_sc_mesh(),
        scratch_types=[
            pltpu.VMEM((1, _C), jnp.int32),
            pltpu.VMEM((1, _C), jnp.int32),
            pltpu.VMEM((_C, d), jnp.float32),
            pltpu.VMEM_SHARED((n_nodes, d), jnp.float32),
            pltpu.SemaphoreType.DMA,
        ],
    )
    def k(x_h, src_h, dst_h, zeros_h, part_h, sidx, didx, rows_v, acc_sh, sem):
        w, c, s = _worker_id()
        r0 = s * rows_per_sub
        pltpu.sync_copy(zeros_h, acc_sh.at[pl.ds(r0, rows_per_sub)])
        plsc.subcore_barrier()

        base = w * epw

        def body(j, carry):
            off = base + j * _C
            pltpu.sync_copy(src_h.at[pl.ds(off, _C)], sidx.at[0])
            pltpu.sync_copy(dst_h.at[pl.ds(off, _C)], didx.at[0])
            pltpu.async_copy(x_h.at[sidx.at[0]], rows_v, sem).wait()
            pltpu.sync_copy(rows_v, acc_sh.at[didx.at[0]], add=True)
            return carry

        lax.fori_loop(0, n_chunks, body, 0)
        plsc.subcore_barrier()
        pltpu.sync_copy(acc_sh.at[pl.ds(r0, rows_per_sub)],
                        part_h.at[c, pl.ds(r0, rows_per_sub)])

    return k(x, src, dst, zeros)


def _tc_prep(feat, dego, degi, L0):
    n, d = feat.shape

    def body(feat_r, dego_r, degi_r, l0_r, x0_r, y0_r, ns_r, nd_r):
        deg_o = dego_r[0, :, 0:1] + dego_r[1, :, 0:1]
        deg_i = degi_r[0, :, 0:1] + degi_r[1, :, 0:1]
        ns = lax.rsqrt(jnp.maximum(deg_o, 1.0))
        nd = lax.rsqrt(jnp.maximum(deg_i, 1.0))
        f = feat_r[...]
        x0_r[...] = f * ns
        y0_r[...] = jnp.dot(f, l0_r[...], preferred_element_type=jnp.float32)
        ns_r[...] = ns
        nd_r[...] = nd

    return pl.pallas_call(
        body,
        out_shape=[
            jax.ShapeDtypeStruct((n, d), jnp.float32),
            jax.ShapeDtypeStruct((n, L0.shape[1]), jnp.float32),
            jax.ShapeDtypeStruct((n, 1), jnp.float32),
            jax.ShapeDtypeStruct((n, 1), jnp.float32),
        ],
    )(feat, dego, degi, L0)


def _tc_mid(part, nd, ns, W, y_prev, L, b):
    n, d = y_prev.shape
    d_next = L.shape[1]
    has_b = b is not None

    def body(part_r, nd_r, ns_r, w_r, y_r, l_r, *rest):
        if has_b:
            b_r, xn_r, yn_r = rest
        else:
            xn_r, yn_r = rest
        agg = (part_r[0] + part_r[1]) * nd_r[...]
        h = jnp.dot(agg, w_r[...], preferred_element_type=jnp.float32) + y_r[...]
        xn_r[...] = h * ns_r[...]
        yn = jnp.dot(h, l_r[...], preferred_element_type=jnp.float32)
        if has_b:
            yn = yn + b_r[...]
        yn_r[...] = yn

    args = [part, nd, ns, W, y_prev, L]
    if has_b:
        args.append(b.reshape(1, -1))
    return pl.pallas_call(
        body,
        out_shape=[
            jax.ShapeDtypeStruct((n, d), jnp.float32),
            jax.ShapeDtypeStruct((n, d_next), jnp.float32),
        ],
    )(*args)


def _tc_final(part, nd, W, y_prev):
    n, d_out = y_prev.shape

    def body(part_r, nd_r, w_r, y_r, out_r):
        agg = (part_r[0] + part_r[1]) * nd_r[...]
        out_r[...] = (jnp.dot(agg, w_r[...], preferred_element_type=jnp.float32)
                      + y_r[...])

    return pl.pallas_call(
        body,
        out_shape=jax.ShapeDtypeStruct((n, d_out), jnp.float32),
    )(part, nd, W, y_prev)


def kernel(feat, edge_index, W0, W1, W2, b2, L0, L1, L2):
    n, d = feat.shape
    src = edge_index[0]
    dst = edge_index[1]

    dego, degi = _degrees(src, dst, n)
    x0, y0, ns, nd = _tc_prep(feat, dego, degi, L0)

    p0 = _spmm(x0, src, dst)
    x1, y1 = _tc_mid(p0, nd, ns, W0, y0, L1, None)

    p1 = _spmm(x1, src, dst)
    x2, y2 = _tc_mid(p1, nd, ns, W1, y1, L2, b2)

    p2 = _spmm(x2, src, dst)
    return _tc_final(p2, nd, W2, y2)


# speedup vs baseline: 4.3743x; 4.3743x over previous
import functools

import jax
import jax.numpy as jnp
from jax import lax
from jax.experimental import pallas as pl
from jax.experimental.pallas import tpu as pltpu
from jax.experimental.pallas import tpu_sc as plsc

_NC = 2
_NS = 16
_NW = _NC * _NS
_C = 80


def _sc_mesh():
    return plsc.VectorSubcoreMesh(core_axis_name="c", subcore_axis_name="s",
                                  num_cores=_NC, num_subcores=_NS)


def _worker_id():
    c = lax.axis_index("c")
    s = lax.axis_index("s")
    return c * _NS + s, c, s


def _degrees(src, dst, n_pad):
    e = src.shape[0]
    assert e % _NS == 0
    epw = e // _NS
    n_chunks = epw // _C
    assert n_chunks * _C == epw
    rows_per_sub = n_pad // _NS
    assert rows_per_sub % 8 == 0

    ones = jnp.ones((_C, 128), dtype=jnp.float32)
    zeros = jnp.zeros((rows_per_sub, 128), dtype=jnp.float32)
    srcdst = jnp.concatenate([src, dst])

    @functools.partial(
        pl.kernel,
        out_type=jax.ShapeDtypeStruct((_NC, n_pad, 128), jnp.float32),
        mesh=_sc_mesh(),
        scratch_types=[
            pltpu.VMEM((1, _C), jnp.int32),
            pltpu.VMEM((_C, 128), jnp.float32),
            pltpu.VMEM_SHARED((n_pad, 128), jnp.float32),
        ],
    )
    def k(sd_h, ones_h, zeros_h, degs_h, didx, ones_v, acc_sh):
        _, c, s = _worker_id()
        r0 = s * rows_per_sub
        pltpu.sync_copy(zeros_h, acc_sh.at[pl.ds(r0, rows_per_sub)])
        pltpu.sync_copy(ones_h, ones_v)
        plsc.subcore_barrier()

        base = c * e + s * epw

        def body(j, carry):
            off = base + j * _C
            pltpu.sync_copy(sd_h.at[pl.ds(off, _C)], didx.at[0])
            pltpu.sync_copy(ones_v, acc_sh.at[didx.at[0]], add=True)
            return carry

        lax.fori_loop(0, n_chunks, body, 0)
        plsc.subcore_barrier()
        pltpu.sync_copy(acc_sh.at[pl.ds(r0, rows_per_sub)],
                        degs_h.at[c, pl.ds(r0, rows_per_sub)])

    return k(srcdst, ones, zeros)


def _spmm(x, src, dst, n_pad):
    n_nodes, d = x.shape
    e = src.shape[0]
    epw = e // _NW
    n_chunks = epw // _C
    rows_per_sub = n_pad // _NS
    zeros = jnp.zeros((rows_per_sub, d), dtype=jnp.float32)

    @functools.partial(
        pl.kernel,
        out_type=jax.ShapeDtypeStruct((_NC, n_pad, d), jnp.float32),
        mesh=_sc_mesh(),
        scratch_types=[
            pltpu.VMEM((1, _C), jnp.int32),
            pltpu.VMEM((1, _C), jnp.int32),
            pltpu.VMEM((_C, d), jnp.float32),
            pltpu.VMEM_SHARED((n_pad, d), jnp.float32),
            pltpu.SemaphoreType.DMA,
        ],
    )
    def k(x_h, src_h, dst_h, zeros_h, part_h, sidx, didx, rows_v, acc_sh, sem):
        w, c, s = _worker_id()
        r0 = s * rows_per_sub
        pltpu.sync_copy(zeros_h, acc_sh.at[pl.ds(r0, rows_per_sub)])
        plsc.subcore_barrier()

        base = w * epw

        def body(j, carry):
            off = base + j * _C
            pltpu.sync_copy(src_h.at[pl.ds(off, _C)], sidx.at[0])
            pltpu.sync_copy(dst_h.at[pl.ds(off, _C)], didx.at[0])
            pltpu.async_copy(x_h.at[sidx.at[0]], rows_v, sem).wait()
            pltpu.sync_copy(rows_v, acc_sh.at[didx.at[0]], add=True)
            return carry

        lax.fori_loop(0, n_chunks, body, 0)
        plsc.subcore_barrier()
        pltpu.sync_copy(acc_sh.at[pl.ds(r0, rows_per_sub)],
                        part_h.at[c, pl.ds(r0, rows_per_sub)])

    return k(x, src, dst, zeros)


def _tc_prep(feat, degs, L0):
    n, d = feat.shape

    def body(feat_r, degs_r, l0_r, x0_r, y0_r, ns_r, nd_r):
        deg_o = degs_r[0, :n, 0:1]
        deg_i = degs_r[1, :n, 0:1]
        ns = lax.rsqrt(jnp.maximum(deg_o, 1.0))
        nd = lax.rsqrt(jnp.maximum(deg_i, 1.0))
        f = feat_r[...]
        x0_r[...] = f * ns
        y0_r[...] = jnp.dot(f, l0_r[...], preferred_element_type=jnp.float32)
        ns_r[...] = ns
        nd_r[...] = nd

    return pl.pallas_call(
        body,
        out_shape=[
            jax.ShapeDtypeStruct((n, d), jnp.float32),
            jax.ShapeDtypeStruct((n, L0.shape[1]), jnp.float32),
            jax.ShapeDtypeStruct((n, 1), jnp.float32),
            jax.ShapeDtypeStruct((n, 1), jnp.float32),
        ],
    )(feat, degs, L0)


def _tc_mid(part, nd, ns, W, y_prev, L, b):
    n, d = y_prev.shape
    d_next = L.shape[1]
    has_b = b is not None

    def body(part_r, nd_r, ns_r, w_r, y_r, l_r, *rest):
        if has_b:
            b_r, xn_r, yn_r = rest
        else:
            xn_r, yn_r = rest
        agg = (part_r[0, :n] + part_r[1, :n]) * nd_r[...]
        h = jnp.dot(agg, w_r[...], preferred_element_type=jnp.float32) + y_r[...]
        xn_r[...] = h * ns_r[...]
        yn = jnp.dot(h, l_r[...], preferred_element_type=jnp.float32)
        if has_b:
            yn = yn + b_r[...]
        yn_r[...] = yn

    args = [part, nd, ns, W, y_prev, L]
    if has_b:
        args.append(b.reshape(1, -1))
    return pl.pallas_call(
        body,
        out_shape=[
            jax.ShapeDtypeStruct((n, d), jnp.float32),
            jax.ShapeDtypeStruct((n, d_next), jnp.float32),
        ],
    )(*args)


def _tc_final(part, nd, W, y_prev):
    n, d_out = y_prev.shape

    def body(part_r, nd_r, w_r, y_r, out_r):
        agg = (part_r[0, :n] + part_r[1, :n]) * nd_r[...]
        out_r[...] = (jnp.dot(agg, w_r[...], preferred_element_type=jnp.float32)
                      + y_r[...])

    return pl.pallas_call(
        body,
        out_shape=jax.ShapeDtypeStruct((n, d_out), jnp.float32),
    )(part, nd, W, y_prev)


def kernel(feat, edge_index, W0, W1, W2, b2, L0, L1, L2):
    n, d = feat.shape
    n_pad = ((n + 127) // 128) * 128
    src = edge_index[0]
    dst = edge_index[1]

    degs = _degrees(src, dst, n_pad)
    x0, y0, ns, nd = _tc_prep(feat, degs, L0)

    p0 = _spmm(x0, src, dst, n_pad)
    x1, y1 = _tc_mid(p0, nd, ns, W0, y0, L1, None)

    p1 = _spmm(x1, src, dst, n_pad)
    x2, y2 = _tc_mid(p1, nd, ns, W1, y1, L2, b2)

    p2 = _spmm(x2, src, dst, n_pad)
    return _tc_final(p2, nd, W2, y2)
